# Initial kernel scaffold; baseline (speedup 1.0000x reference)
import jax
import jax.numpy as jnp
from jax.experimental import pallas as pl


def kernel(x, W, b):
    raise NotImplementedError("write your pallas kernel here")


# speedup vs baseline: 1.0039x; 1.0039x over previous
import jax
import jax.numpy as jnp
from jax import lax
from jax.experimental import pallas as pl
from jax.experimental.pallas import tpu as pltpu


def _linear_kernel(x_ref, w_ref, b_ref, o_ref):
    w_bf = w_ref[...].astype(jnp.bfloat16)
    acc = lax.dot_general(
        x_ref[...], w_bf,
        dimension_numbers=(((1,), (1,)), ((), ())),
        preferred_element_type=jnp.float32,
    )
    o_ref[...] = acc + b_ref[...]


def _pick_bn(n):
    for bn in (256, 128):
        if n % bn == 0:
            return bn
    return n


def kernel(x, W, b):
    M, K = x.shape
    N = W.shape[0]
    BN = _pick_bn(N)

    x_bf = x.astype(jnp.bfloat16)
    b2 = b.reshape(1, N)

    out = pl.pallas_call(
        _linear_kernel,
        grid=(N // BN,),
        in_specs=[
            pl.BlockSpec((M, K), lambda i: (0, 0)),
            pl.BlockSpec((BN, K), lambda i: (i, 0)),
            pl.BlockSpec((1, BN), lambda i: (0, i)),
        ],
        out_specs=pl.BlockSpec((M, BN), lambda i: (0, i)),
        out_shape=jax.ShapeDtypeStruct((M, N), jnp.float32),
        compiler_params=pltpu.CompilerParams(
            dimension_semantics=("parallel",),
        ),
    )(x_bf, W, b2)
    return out


# speedup vs baseline: 1.0048x; 1.0009x over previous
import jax
import jax.numpy as jnp
from jax import lax
from jax.experimental import pallas as pl
from jax.experimental.pallas import tpu as pltpu


def _linear_kernel(x_ref, w_ref, b_ref, o_ref):
    w_bf = w_ref[...].astype(jnp.bfloat16)
    acc = lax.dot_general(
        x_ref[...], w_bf,
        dimension_numbers=(((1,), (1,)), ((), ())),
        preferred_element_type=jnp.float32,
    )
    o_ref[...] = acc + b_ref[...]


def _pick_bn(n):
    for bn in (256, 128):
        if n % bn == 0:
            return bn
    return n


def kernel(x, W, b):
    M, K = x.shape
    N = W.shape[0]
    BN = _pick_bn(N)

    x_bf = x.astype(jnp.bfloat16)
    b2 = b.reshape(1, N)

    out = pl.pallas_call(
        _linear_kernel,
        grid=(N // BN,),
        in_specs=[
            pl.BlockSpec((M, K), lambda i: (0, 0)),
            pl.BlockSpec((BN, K), lambda i: (i, 0)),
            pl.BlockSpec((1, BN), lambda i: (0, i)),
        ],
        out_specs=pl.BlockSpec((M, BN), lambda i: (0, i)),
        out_shape=jax.ShapeDtypeStruct((M, N), jnp.float32),
        compiler_params=pltpu.CompilerParams(
            dimension_semantics=("arbitrary",),
        ),
    )(x_bf, W, b2)
    return out


# speedup vs baseline: 1.0393x; 1.0344x over previous
import jax
import jax.numpy as jnp
from jax import lax
from jax.experimental import pallas as pl
from jax.experimental.pallas import tpu as pltpu


def _linear_kernel(x_ref, w_ref, b_ref, o_ref):
    w_bf = w_ref[...].astype(jnp.bfloat16)
    m = x_ref.shape[0]
    bm = m // 2
    for mo in (0, bm):
        acc = lax.dot_general(
            x_ref[pl.ds(mo, bm), :], w_bf,
            dimension_numbers=(((1,), (1,)), ((), ())),
            preferred_element_type=jnp.float32,
        )
        o_ref[pl.ds(mo, bm), :] = acc + b_ref[...]


def _pick_bn(n):
    for bn in (256, 128):
        if n % bn == 0:
            return bn
    return n


def kernel(x, W, b):
    M, K = x.shape
    N = W.shape[0]
    BN = _pick_bn(N)

    x_bf = x.astype(jnp.bfloat16)
    b2 = b.reshape(1, N)

    out = pl.pallas_call(
        _linear_kernel,
        grid=(N // BN,),
        in_specs=[
            pl.BlockSpec((M, K), lambda i: (0, 0)),
            pl.BlockSpec((BN, K), lambda i: (i, 0)),
            pl.BlockSpec((1, BN), lambda i: (0, i)),
        ],
        out_specs=pl.BlockSpec((M, BN), lambda i: (0, i)),
        out_shape=jax.ShapeDtypeStruct((M, N), jnp.float32),
        compiler_params=pltpu.CompilerParams(
            dimension_semantics=("arbitrary",),
        ),
    )(x_bf, W, b2)
    return out
